# Initial kernel scaffold; baseline (speedup 1.0000x reference)
import jax
import jax.numpy as jnp
from jax.experimental import pallas as pl


def kernel(x, top_k, W, b):
    raise NotImplementedError("write your pallas kernel here")


# speedup vs baseline: 1.4806x; 1.4806x over previous
import functools

import jax
import jax.numpy as jnp
from jax import lax
from jax.experimental import pallas as pl
from jax.experimental.pallas import tpu as pltpu
from jax.experimental.pallas import tpu_sc as plsc


def _permute_body(x_ref, p_ref, o_ref):
    o_ref[...] = jnp.dot(x_ref[...], p_ref[...],
                         preferred_element_type=jnp.float32)


def _attn_body(tokE_ref, tokO_ref, par_ref, pe4s_ref, peel_ref, peol_ref,
               w_ref, b_ref, ve_ref, vo_ref):
    hd = tokE_ref.shape[1]
    tokE = tokE_ref[0]
    tokO = tokO_ref[0]
    tok = jnp.where(par_ref[0] != 0, tokO, tokE)
    qkvT = jnp.dot(w_ref[...], tok,
                   preferred_element_type=jnp.float32) + b_ref[...]
    qT = qkvT[0:hd]
    kT = qkvT[hd:2 * hd]
    vT = qkvT[2 * hd:3 * hd]
    attn = lax.dot_general(kT, qT, (((0,), (0,)), ((), ())),
                           preferred_element_type=jnp.float32)
    attn = attn - jnp.max(attn, axis=-1, keepdims=True)
    attn = jnp.exp(attn)
    attn = attn / jnp.sum(attn, axis=-1, keepdims=True)
    outT = lax.dot_general(vT, attn, (((1,), (1,)), ((), ())),
                           preferred_element_type=jnp.float32)
    me = (pe4s_ref[0] == peel_ref[0]).astype(jnp.float32)
    mo = (pe4s_ref[0] == peol_ref[0]).astype(jnp.float32)
    ve_ref[0] = tokE + lax.dot_general(
        outT, me, (((1,), (0,)), ((), ())),
        preferred_element_type=jnp.float32)
    vo_ref[0] = tokO + lax.dot_general(
        outT, mo, (((1,), (0,)), ((), ())),
        preferred_element_type=jnp.float32)


def _sc_worker_id():
    info = plsc.get_sparse_core_info()
    return lax.axis_index("s") * info.num_cores + lax.axis_index("c")


def _gather_body(yt_hbm, idx_hbm, tok_hbm, idx_v, rows_v, sem):
    rpw = idx_v.shape[0]
    base = _sc_worker_id() * rpw
    pltpu.sync_copy(idx_hbm.at[pl.ds(base, rpw)], idx_v)
    copies = [pltpu.async_copy(yt_hbm.at[idx_v.at[j]], rows_v.at[j], sem)
              for j in range(rpw)]
    for c in copies:
        c.wait()
    pltpu.sync_copy(rows_v, tok_hbm.at[pl.ds(base, rpw)])


def _scatter_body(yt_hbm, idx_hbm, vals_hbm, idx_v, vals_v, sem):
    rpw = idx_v.shape[0]
    base = _sc_worker_id() * rpw
    pltpu.sync_copy(idx_hbm.at[pl.ds(base, rpw)], idx_v)
    pltpu.sync_copy(vals_hbm.at[pl.ds(base, rpw)], vals_v)
    copies = [pltpu.async_copy(vals_v.at[j], yt_hbm.at[idx_v.at[j]], sem)
              for j in range(rpw)]
    for c in copies:
        c.wait()


def kernel(x, top_k, W, b):
    f32 = jnp.float32
    i32 = jnp.int32
    B, C, H, Wd = x.shape
    hd = W.shape[1]
    nh = C // hd
    Ph, Pw = H // 2, Wd // 2
    P = Ph * Pw
    K = top_k.shape[-1]
    T = 4 * K
    NBH = B * nh
    NR = NBH * hd
    NROWS = B * C * Ph

    u = jnp.arange(2 * Wd, dtype=i32)
    v_of_u = 4 * ((u % Wd) // 2) + 2 * (u // Wd) + (u % 2)
    perm = (v_of_u[:, None] ==
            jnp.arange(2 * Wd, dtype=i32)[None, :]).astype(f32)

    rows_blk = 896
    grid1 = NROWS // rows_blk
    x2 = x.reshape(NROWS, 2 * Wd)
    y2 = pl.pallas_call(
        _permute_body,
        grid=(grid1,),
        in_specs=[
            pl.BlockSpec((rows_blk, 2 * Wd), lambda i: (i, 0)),
            pl.BlockSpec((2 * Wd, 2 * Wd), lambda i: (0, 0)),
        ],
        out_specs=pl.BlockSpec((rows_blk, 2 * Wd), lambda i: (i, 0)),
        out_shape=jax.ShapeDtypeStruct((NROWS, 2 * Wd), f32),
    )(x2, perm)
    yt8 = y2.reshape(NR * P // 2, 8)

    tk = top_k.astype(i32)
    idx8 = (jnp.arange(NR, dtype=i32).reshape(NBH, hd, 1) * (P // 2)
            + (tk.reshape(NBH, 1, K) >> 1)).reshape(NR, K)

    info = plsc.get_sparse_core_info()
    nw = info.num_cores * info.num_subcores
    rpw = NR // nw
    mesh = plsc.VectorSubcoreMesh(core_axis_name="c", subcore_axis_name="s")
    sc_params = pltpu.CompilerParams(use_tc_tiling_on_sc=False)
    tok8 = pl.kernel(
        _gather_body,
        out_type=jax.ShapeDtypeStruct((NR, K, 8), f32),
        mesh=mesh,
        compiler_params=sc_params,
        scratch_types=[
            pltpu.VMEM((rpw, K), i32),
            pltpu.VMEM((rpw, K, 8), f32),
            pltpu.SemaphoreType.DMA,
        ],
    )(yt8, idx8)

    tok8 = tok8.reshape(NBH, hd, K, 8)
    tokE = tok8[..., 0:4].reshape(NBH, hd, T)
    tokO = tok8[..., 4:8].reshape(NBH, hd, T)
    c4 = jnp.arange(4, dtype=i32)
    pe4 = (tk.reshape(NBH, K, 1) * 4 + c4).reshape(NBH, T)
    peE = ((tk.reshape(NBH, K, 1) >> 1) * 8 + c4).reshape(NBH, T)
    peO = peE + 4
    par = jnp.broadcast_to((tk.reshape(NBH, K, 1) & 1),
                           (NBH, K, 4)).reshape(NBH, 1, T)
    ve, vo = pl.pallas_call(
        _attn_body,
        grid=(NBH,),
        in_specs=[
            pl.BlockSpec((1, hd, T), lambda i: (i, 0, 0)),
            pl.BlockSpec((1, hd, T), lambda i: (i, 0, 0)),
            pl.BlockSpec((1, 1, T), lambda i: (i, 0, 0)),
            pl.BlockSpec((1, T, 1), lambda i: (i, 0, 0)),
            pl.BlockSpec((1, 1, T), lambda i: (i, 0, 0)),
            pl.BlockSpec((1, 1, T), lambda i: (i, 0, 0)),
            pl.BlockSpec((3 * hd, hd), lambda i: (0, 0)),
            pl.BlockSpec((3 * hd, 1), lambda i: (0, 0)),
        ],
        out_specs=[
            pl.BlockSpec((1, hd, T), lambda i: (i, 0, 0)),
            pl.BlockSpec((1, hd, T), lambda i: (i, 0, 0)),
        ],
        out_shape=[
            jax.ShapeDtypeStruct((NBH, hd, T), f32),
            jax.ShapeDtypeStruct((NBH, hd, T), f32),
        ],
    )(tokE, tokO, par, pe4.reshape(NBH, T, 1),
      peE.reshape(NBH, 1, T), peO.reshape(NBH, 1, T),
      W, b.reshape(3 * hd, 1))

    ve4 = ve.reshape(NR, K, 1, 4)
    vo4 = vo.reshape(NR, K, 1, 4)
    vals8 = jnp.concatenate([ve4, vo4], axis=2).reshape(NR, K, 8)

    ytr = jax.new_ref(yt8)
    pl.kernel(
        _scatter_body,
        out_type=(),
        mesh=mesh,
        compiler_params=sc_params,
        scratch_types=[
            pltpu.VMEM((rpw, K), i32),
            pltpu.VMEM((rpw, K, 8), f32),
            pltpu.SemaphoreType.DMA,
        ],
    )(ytr, idx8, vals8)
    return ytr[...].reshape(B, C, H, Wd)
